# baseline (speedup 1.0000x reference)
import functools

import jax
import jax.numpy as jnp
from jax import lax
from jax.experimental import pallas as pl
from jax.experimental.pallas import tpu as pltpu
from jax.experimental.pallas import tpu_sc as plsc

NUM_CORES = 2
NUM_SUBCORES = 16
NUM_WORKERS = NUM_CORES * NUM_SUBCORES
LANES = 16
EMBED = 64
HALF = 256
IDX_CHUNK = 128
TS = 17


def _sc_recommender(b_per_w):
    mesh = plsc.VectorSubcoreMesh(
        core_axis_name="c", subcore_axis_name="s", num_cores=NUM_CORES
    )

    @functools.partial(
        pl.kernel,
        mesh=mesh,
        compiler_params=pltpu.CompilerParams(
            needs_layout_passes=False, use_tc_tiling_on_sc=True
        ),
        out_type=jax.ShapeDtypeStruct((b_per_w * NUM_WORKERS,), jnp.float32),
        scratch_types=[
            pltpu.VMEM((b_per_w + LANES,), jnp.int32),
            pltpu.VMEM((b_per_w + LANES,), jnp.int32),
            pltpu.VMEM((HALF, EMBED), jnp.float32),
            pltpu.VMEM((HALF, EMBED), jnp.float32),
            pltpu.VMEM((b_per_w,), jnp.float32),
            pltpu.VMEM((b_per_w,), jnp.float32),
            pltpu.VMEM((b_per_w,), jnp.float32),
            pltpu.VMEM((LANES * TS,), jnp.float32),
            pltpu.SemaphoreType.DMA,
            pltpu.SemaphoreType.DMA,
            pltpu.SemaphoreType.DMA,
        ],
    )
    def body(uidx_hbm, midx_hbm, uemb_hbm, memb_hbm, ubias_hbm, mbias_hbm,
             out_hbm, uidx_v, midx_v, urows_v, mrows_v, ubias_v, mbias_v,
             out_v, ts_v, semu, semm, semb):
        wid = lax.axis_index("s") * NUM_CORES + lax.axis_index("c")
        base = wid * b_per_w

        pltpu.sync_copy(uidx_hbm.at[pl.ds(base, b_per_w)],
                        uidx_v.at[pl.ds(0, b_per_w)])
        pltpu.sync_copy(midx_hbm.at[pl.ds(base, b_per_w)],
                        midx_v.at[pl.ds(0, b_per_w)])

        bias_copies = []
        for j in range(b_per_w // IDX_CHUNK):
            sl = pl.ds(j * IDX_CHUNK, IDX_CHUNK)
            bias_copies.append(pltpu.async_copy(
                ubias_hbm.at[uidx_v.at[sl]], ubias_v.at[sl], semb))
            bias_copies.append(pltpu.async_copy(
                mbias_hbm.at[midx_v.at[sl]], mbias_v.at[sl], semb))

        lane17 = lax.iota(jnp.int32, LANES) * TS

        def fire_row(r, h0):
            urow = uidx_v[pl.ds(h0 + r, LANES)][0]
            mrow = midx_v[pl.ds(h0 + r, LANES)][0]
            pltpu.async_copy(uemb_hbm.at[pl.ds(urow, 1), :],
                             urows_v.at[pl.ds(r, 1), :], semu)
            pltpu.async_copy(memb_hbm.at[pl.ds(mrow, 1), :],
                             mrows_v.at[pl.ds(r, 1), :], semm)
            return h0

        def drain_row(r, _):
            pltpu.make_async_copy(uemb_hbm.at[pl.ds(0, 1), :],
                                  urows_v.at[pl.ds(0, 1), :], semu).wait()
            pltpu.make_async_copy(memb_hbm.at[pl.ds(0, 1), :],
                                  mrows_v.at[pl.ds(0, 1), :], semm).wait()
            return 0

        def group(g, h0):
            row0 = g * LANES
            for k in range(LANES):
                r = row0 + k
                s = urows_v[r, pl.ds(0, LANES)] * mrows_v[r, pl.ds(0, LANES)]
                for c in range(1, EMBED // LANES):
                    s = s + (urows_v[r, pl.ds(c * LANES, LANES)]
                             * mrows_v[r, pl.ds(c * LANES, LANES)])
                plsc.store_scatter(ts_v, [lane17 + k], s)
            acc = ts_v[pl.ds(0, LANES)]
            for i in range(1, LANES):
                acc = acc + ts_v[pl.ds(i * TS, LANES)]
            out_v[pl.ds(h0 + row0, LANES)] = acc
            return h0

        for half in range(b_per_w // HALF):
            h0 = half * HALF
            lax.fori_loop(0, HALF, fire_row, h0)
            lax.fori_loop(0, HALF, drain_row, 0)
            lax.fori_loop(0, HALF // LANES, group, h0)

        for c in bias_copies:
            c.wait()

        def finish(g, _):
            sl = pl.ds(g * LANES, LANES)
            x = out_v[sl] + ubias_v[sl] + mbias_v[sl]
            out_v[sl] = 1.0 / (1.0 + jnp.exp(-x))
            return 0

        lax.fori_loop(0, b_per_w // LANES, finish, 0)
        pltpu.sync_copy(out_v, out_hbm.at[pl.ds(base, b_per_w)])

    return body


TCOLS = 1024


def _tc_transpose(n_ent):
    def body(src_ref, dst_ref):
        dst_ref[...] = src_ref[...].T

    return pl.pallas_call(
        body,
        grid=(pl.cdiv(n_ent, TCOLS),),
        in_specs=[pl.BlockSpec((EMBED, TCOLS), lambda i: (0, i))],
        out_specs=pl.BlockSpec((TCOLS, EMBED), lambda i: (i, 0)),
        out_shape=jax.ShapeDtypeStruct((n_ent, EMBED), jnp.float32),
    )


def kernel(inputs, user_emb, movie_emb, user_bias_tab, movie_bias_tab):
    batch = inputs.shape[0]
    b_per_w = batch // NUM_WORKERS
    user_idx = inputs[:, 0]
    movie_idx = inputs[:, 1]
    tr = _tc_transpose(user_emb.shape[0])
    uemb_rm = tr(user_emb.T)
    memb_rm = tr(movie_emb.T)
    fn = _sc_recommender(b_per_w)
    out = fn(user_idx, movie_idx, uemb_rm, memb_rm,
             user_bias_tab.T[0], movie_bias_tab.T[0])
    return out.reshape(batch, 1)


# speedup vs baseline: 1.6355x; 1.6355x over previous
import functools

import jax
import jax.numpy as jnp
from jax import lax
from jax.experimental import pallas as pl
from jax.experimental.pallas import tpu as pltpu
from jax.experimental.pallas import tpu_sc as plsc

NUM_CORES = 2
NUM_SUBCORES = 16
NUM_WORKERS = NUM_CORES * NUM_SUBCORES
LANES = 16
EMBED = 64
HALF = 256
IDX_CHUNK = 128
TS = 17


def _sc_recommender(b_per_w):
    mesh = plsc.VectorSubcoreMesh(
        core_axis_name="c", subcore_axis_name="s", num_cores=NUM_CORES
    )

    @functools.partial(
        pl.kernel,
        mesh=mesh,
        compiler_params=pltpu.CompilerParams(
            needs_layout_passes=False, use_tc_tiling_on_sc=True
        ),
        out_type=jax.ShapeDtypeStruct((b_per_w * NUM_WORKERS,), jnp.float32),
        scratch_types=[
            pltpu.VMEM((b_per_w + LANES,), jnp.int32),
            pltpu.VMEM((b_per_w + LANES,), jnp.int32),
            pltpu.VMEM((HALF, EMBED), jnp.float32),
            pltpu.VMEM((HALF, EMBED), jnp.float32),
            pltpu.VMEM((b_per_w,), jnp.float32),
            pltpu.VMEM((b_per_w,), jnp.float32),
            pltpu.VMEM((b_per_w,), jnp.float32),
            pltpu.VMEM((LANES * TS,), jnp.float32),
            pltpu.SemaphoreType.DMA,
            pltpu.SemaphoreType.DMA,
            pltpu.SemaphoreType.DMA,
        ],
    )
    def body(uidx_hbm, midx_hbm, uemb_hbm, memb_hbm, ubias_hbm, mbias_hbm,
             out_hbm, uidx_v, midx_v, urows_v, mrows_v, ubias_v, mbias_v,
             out_v, ts_v, semu, semm, semb):
        wid = lax.axis_index("s") * NUM_CORES + lax.axis_index("c")
        base = wid * b_per_w

        pltpu.sync_copy(uidx_hbm.at[pl.ds(base, b_per_w)],
                        uidx_v.at[pl.ds(0, b_per_w)])
        pltpu.sync_copy(midx_hbm.at[pl.ds(base, b_per_w)],
                        midx_v.at[pl.ds(0, b_per_w)])

        bias_copies = []
        for j in range(b_per_w // IDX_CHUNK):
            sl = pl.ds(j * IDX_CHUNK, IDX_CHUNK)
            bias_copies.append(pltpu.async_copy(
                ubias_hbm.at[uidx_v.at[sl]], ubias_v.at[sl], semb))
            bias_copies.append(pltpu.async_copy(
                mbias_hbm.at[midx_v.at[sl]], mbias_v.at[sl], semb))

        lane17 = lax.iota(jnp.int32, LANES) * TS

        def fire_row(r, h0):
            urow = uidx_v[pl.ds(h0 + r, LANES)][0]
            mrow = midx_v[pl.ds(h0 + r, LANES)][0]
            pltpu.async_copy(uemb_hbm.at[pl.ds(urow, 1), :],
                             urows_v.at[pl.ds(r, 1), :], semu)
            pltpu.async_copy(memb_hbm.at[pl.ds(mrow, 1), :],
                             mrows_v.at[pl.ds(r, 1), :], semm)
            return h0

        def drain_row(r, _):
            pltpu.make_async_copy(uemb_hbm.at[pl.ds(0, 1), :],
                                  urows_v.at[pl.ds(0, 1), :], semu).wait()
            pltpu.make_async_copy(memb_hbm.at[pl.ds(0, 1), :],
                                  mrows_v.at[pl.ds(0, 1), :], semm).wait()
            return 0

        def group(g, h0):
            row0 = g * LANES
            for k in range(LANES):
                r = row0 + k
                s = urows_v[r, pl.ds(0, LANES)] * mrows_v[r, pl.ds(0, LANES)]
                for c in range(1, EMBED // LANES):
                    s = s + (urows_v[r, pl.ds(c * LANES, LANES)]
                             * mrows_v[r, pl.ds(c * LANES, LANES)])
                plsc.store_scatter(ts_v, [lane17 + k], s)
            acc = ts_v[pl.ds(0, LANES)]
            for i in range(1, LANES):
                acc = acc + ts_v[pl.ds(i * TS, LANES)]
            out_v[pl.ds(h0 + row0, LANES)] = acc
            return h0

        for half in range(b_per_w // HALF):
            h0 = half * HALF
            lax.fori_loop(0, HALF, fire_row, h0)
            lax.fori_loop(0, HALF, drain_row, 0)
            lax.fori_loop(0, HALF // LANES, group, h0)

        for c in bias_copies:
            c.wait()

        def finish(g, _):
            sl = pl.ds(g * LANES, LANES)
            x = out_v[sl] + ubias_v[sl] + mbias_v[sl]
            out_v[sl] = 1.0 / (1.0 + jnp.exp(-x))
            return 0

        lax.fori_loop(0, b_per_w // LANES, finish, 0)
        pltpu.sync_copy(out_v, out_hbm.at[pl.ds(base, b_per_w)])

    return body


def kernel(inputs, user_emb, movie_emb, user_bias_tab, movie_bias_tab):
    batch = inputs.shape[0]
    b_per_w = batch // NUM_WORKERS
    user_idx = inputs[:, 0]
    movie_idx = inputs[:, 1]
    fn = _sc_recommender(b_per_w)
    out = fn(user_idx, movie_idx, user_emb, movie_emb,
             user_bias_tab.T[0], movie_bias_tab.T[0])
    return out.reshape(batch, 1)
